# baseline (speedup 1.0000x reference)
import functools

import jax
import jax.numpy as jnp
from jax import lax
from jax.experimental import pallas as pl
from jax.experimental.pallas import tpu as pltpu
from jax.experimental.pallas import tpu_sc as plsc

B = 4
T = 8192
D = 1024

_info = plsc.get_sparse_core_info()
NC = _info.num_cores
NS = _info.num_subcores
NW = NC * NS
LANES = 16

TPW = T // NW
CT = 32
CHUNK = CT * D
N_TC = TPW // CT
VITERS = CHUNK // LANES

_mesh = plsc.VectorSubcoreMesh(core_axis_name="c", subcore_axis_name="s")


@functools.partial(
    pl.kernel,
    out_type=jax.ShapeDtypeStruct((B * T * D,), jnp.float32),
    mesh=_mesh,
    scratch_types=[
        pltpu.VMEM((CHUNK,), jnp.float32),
        pltpu.VMEM((CHUNK,), jnp.float32),
    ],
)
def _pos_add(enc_hbm, pos_hbm, out_hbm, enc_v, pos_v):
    wid = lax.axis_index("s") * NC + lax.axis_index("c")
    tok_base = wid * TPW

    def tc_body(tc, _):
        tok0 = tok_base + tc * CT
        pltpu.sync_copy(pos_hbm.at[pl.ds(tok0 * D, CHUNK)], pos_v)
        for b in range(B):
            off = b * (T * D) + tok0 * D
            pltpu.sync_copy(enc_hbm.at[pl.ds(off, CHUNK)], enc_v)

            def vbody(k, _):
                s = pl.ds(k * LANES, LANES)
                enc_v[s] = enc_v[s] + pos_v[s]
                return 0

            lax.fori_loop(0, VITERS, vbody, 0)
            pltpu.sync_copy(enc_v, out_hbm.at[pl.ds(off, CHUNK)])
        return 0

    lax.fori_loop(0, N_TC, tc_body, 0)


def kernel(encode_token, pos_table):
    enc = encode_token.reshape(-1)
    pos = pos_table.reshape(-1)
    out = _pos_add(enc, pos)
    return out.reshape(B, T, D)


# speedup vs baseline: 1.9492x; 1.9492x over previous
import functools

import jax
import jax.numpy as jnp
from jax import lax
from jax.experimental import pallas as pl
from jax.experimental.pallas import tpu as pltpu
from jax.experimental.pallas import tpu_sc as plsc

B = 4
T = 8192
D = 1024

_info = plsc.get_sparse_core_info()
NC = _info.num_cores
NS = _info.num_subcores
NW = NC * NS
LANES = 16

TPW = T // NW
CT = 4
CHUNK = CT * D
N_CH = TPW // CT
K = 4
GROUPS = CHUNK // LANES
U = 8

_mesh = plsc.VectorSubcoreMesh(core_axis_name="c", subcore_axis_name="s")


@functools.partial(
    pl.kernel,
    out_type=jax.ShapeDtypeStruct((B, T * D), jnp.float32),
    mesh=_mesh,
    scratch_types=(
        [pltpu.VMEM((B, CHUNK), jnp.float32) for _ in range(K)]
        + [pltpu.VMEM((CHUNK,), jnp.float32) for _ in range(K)]
        + [pltpu.SemaphoreType.DMA for _ in range(2 * K)]
    ),
)
def _pos_add(enc_hbm, pos_hbm, out_hbm, *scratch):
    enc_v = scratch[:K]
    pos_v = scratch[K:2 * K]
    in_sem = scratch[2 * K:3 * K]
    out_sem = scratch[3 * K:4 * K]

    wid = lax.axis_index("s") * NC + lax.axis_index("c")
    word_base = wid * TPW * D

    def issue_in(g, q):
        off = word_base + g * CHUNK
        pltpu.async_copy(enc_hbm.at[:, pl.ds(off, CHUNK)], enc_v[q], in_sem[q])
        pltpu.async_copy(pos_hbm.at[pl.ds(off, CHUNK)], pos_v[q], in_sem[q])

    def drain_in(g, q):
        off = word_base + g * CHUNK
        pltpu.make_async_copy(
            enc_hbm.at[:, pl.ds(off, CHUNK)], enc_v[q], in_sem[q]).wait()
        pltpu.make_async_copy(
            pos_hbm.at[pl.ds(off, CHUNK)], pos_v[q], in_sem[q]).wait()

    def issue_out(g, q):
        off = word_base + g * CHUNK
        pltpu.async_copy(enc_v[q], out_hbm.at[:, pl.ds(off, CHUNK)], out_sem[q])

    def drain_out(g, q):
        off = word_base + g * CHUNK
        pltpu.make_async_copy(
            enc_v[q], out_hbm.at[:, pl.ds(off, CHUNK)], out_sem[q]).wait()

    issue_in(0, 0)

    def j_body(j, _):
        for k in range(K):
            g = j * K + k
            q = (k + 1) % K

            @pl.when(g + 1 - K >= 0)
            def _():
                drain_out(g + 1 - K, q)

            @pl.when(g + 1 < N_CH)
            def _():
                issue_in(g + 1, q)

            drain_in(g, k)

            def vbody(kk, _):
                base = kk * (U * LANES)
                for u in range(U):
                    s = pl.ds(base + u * LANES, LANES)
                    pvec = pos_v[k][s]
                    for b in range(B):
                        enc_v[k][b, s] = enc_v[k][b, s] + pvec
                return 0

            lax.fori_loop(0, GROUPS // U, vbody, 0)
            issue_out(g, k)
        return 0

    lax.fori_loop(0, N_CH // K, j_body, 0)

    for g in range(N_CH - K + 1, N_CH):
        drain_out(g, g % K)


def kernel(encode_token, pos_table):
    enc = encode_token.reshape(B, T * D)
    pos = pos_table.reshape(-1)
    out = _pos_add(enc, pos)
    return out.reshape(B, T, D)


# speedup vs baseline: 5.4475x; 2.7948x over previous
import functools

import jax
import jax.numpy as jnp
from jax import lax
from jax.experimental import pallas as pl
from jax.experimental.pallas import tpu as pltpu
from jax.experimental.pallas import tpu_sc as plsc

B = 4
T = 8192
D = 1024

_info = plsc.get_sparse_core_info()
NC = _info.num_cores
NS = _info.num_subcores
NW = NC * NS
LANES = 16

TPW = T // NW
CT = 4
N_CH = TPW // CT
K = 4
U = 8

_mesh = plsc.VectorSubcoreMesh(core_axis_name="c", subcore_axis_name="s")


@functools.partial(
    pl.kernel,
    out_type=jax.ShapeDtypeStruct((B, T, D), jnp.float32),
    mesh=_mesh,
    scratch_types=(
        [pltpu.VMEM((B, CT, D), jnp.float32) for _ in range(K)]
        + [pltpu.VMEM((CT, D), jnp.float32) for _ in range(K)]
        + [pltpu.SemaphoreType.DMA for _ in range(2 * K)]
    ),
)
def _pos_add(enc_hbm, pos_hbm, out_hbm, *scratch):
    enc_v = scratch[:K]
    pos_v = scratch[K:2 * K]
    in_sem = scratch[2 * K:3 * K]
    out_sem = scratch[3 * K:4 * K]

    wid = lax.axis_index("s") * NC + lax.axis_index("c")
    tok_base = wid * TPW

    def issue_in(g, q):
        t0 = tok_base + g * CT
        pltpu.async_copy(enc_hbm.at[:, pl.ds(t0, CT), :], enc_v[q], in_sem[q])
        pltpu.async_copy(pos_hbm.at[pl.ds(t0, CT), :], pos_v[q], in_sem[q])

    def drain_in(g, q):
        t0 = tok_base + g * CT
        pltpu.make_async_copy(
            enc_hbm.at[:, pl.ds(t0, CT), :], enc_v[q], in_sem[q]).wait()
        pltpu.make_async_copy(
            pos_hbm.at[pl.ds(t0, CT), :], pos_v[q], in_sem[q]).wait()

    def issue_out(g, q):
        t0 = tok_base + g * CT
        pltpu.async_copy(enc_v[q], out_hbm.at[:, pl.ds(t0, CT), :], out_sem[q])

    def drain_out(g, q):
        t0 = tok_base + g * CT
        pltpu.make_async_copy(
            enc_v[q], out_hbm.at[:, pl.ds(t0, CT), :], out_sem[q]).wait()

    issue_in(0, 0)

    def j_body(j, _):
        for k in range(K):
            g = j * K + k
            q = (k + 1) % K

            @pl.when(g + 1 - K >= 0)
            def _():
                drain_out(g + 1 - K, q)

            @pl.when(g + 1 < N_CH)
            def _():
                issue_in(g + 1, q)

            drain_in(g, k)

            for ct in range(CT):
                def vbody(kk, _, ct=ct):
                    base = kk * (U * LANES)
                    for u in range(U):
                        s = pl.ds(base + u * LANES, LANES)
                        pvec = pos_v[k][ct, s]
                        for b in range(B):
                            enc_v[k][b, ct, s] = enc_v[k][b, ct, s] + pvec
                    return 0

                lax.fori_loop(0, D // (U * LANES), vbody, 0)
            issue_out(g, k)
        return 0

    lax.fori_loop(0, N_CH // K, j_body, 0)

    for g in range(N_CH - K + 1, N_CH):
        drain_out(g, g % K)


def kernel(encode_token, pos_table):
    return _pos_add(encode_token, pos_table)
